# baseline (speedup 1.0000x reference)
import jax
import jax.numpy as jnp
import numpy as np
from jax.experimental import pallas as pl

_NUM_ANCHORS = 3
_NUM_CLASSES = 80
_STRIDE = 16.0
_G = 64
_S = _G * _G
_CH = _NUM_CLASSES + 5
_ANCHORS = ((0.12, 0.16), (0.30, 0.38), (0.72, 0.55))
_AW16 = tuple(float(np.float32(a[0]) * _G * _STRIDE) for a in _ANCHORS)
_AH16 = tuple(float(np.float32(a[1]) * _G * _STRIDE) for a in _ANCHORS)


def _yolo_body(x_ref, boxes_ref, conf_ref, cls_ref):
    a = pl.program_id(1)
    xf = x_ref[0].reshape(_CH, _S)

    head = xf[0:8]
    r = jax.lax.broadcasted_iota(jnp.int32, (8, _S), 0)
    head_act = jnp.where((r == 2) | (r == 3), jnp.exp(head),
                         jax.nn.sigmoid(head))
    tail_act = jax.nn.sigmoid(xf[8:_CH])
    act = jnp.concatenate([head_act, tail_act], axis=0)

    act_t = act.T

    s = jax.lax.broadcasted_iota(jnp.int32, (_S, 1), 0)
    gx = (s & (_G - 1)).astype(jnp.float32)
    gy = (s >> 6).astype(jnp.float32)
    aw16 = jnp.where(a == 0, _AW16[0], jnp.where(a == 1, _AW16[1], _AW16[2]))
    ah16 = jnp.where(a == 0, _AH16[0], jnp.where(a == 1, _AH16[1], _AH16[2]))
    col = jax.lax.broadcasted_iota(jnp.int32, (_S, 4), 1)
    offs = jnp.where(col == 0, gx, jnp.where(col == 1, gy, 0.0))
    scale = jnp.where(col == 0, _STRIDE,
                      jnp.where(col == 1, _STRIDE,
                                jnp.where(col == 2, aw16, ah16)))
    boxes_ref[0] = (act_t[:, 0:4] + offs) * scale
    conf_ref[0] = act_t[:, 4:5]
    cls_ref[0] = act_t[:, 5:_CH]


def kernel(x):
    B = x.shape[0]
    nout = B * _NUM_ANCHORS * _S

    boxes, conf, cls = pl.pallas_call(
        _yolo_body,
        grid=(B, _NUM_ANCHORS),
        in_specs=[pl.BlockSpec((1, _CH, _G, _G), lambda b, a: (b, a, 0, 0))],
        out_specs=[
            pl.BlockSpec((1, _S, 4), lambda b, a: (b, a, 0)),
            pl.BlockSpec((1, _S, 1), lambda b, a: (b, a, 0)),
            pl.BlockSpec((1, _S, _NUM_CLASSES), lambda b, a: (b, a, 0)),
        ],
        out_shape=[
            jax.ShapeDtypeStruct((B, _NUM_ANCHORS * _S, 4), jnp.float32),
            jax.ShapeDtypeStruct((B, _NUM_ANCHORS * _S, 1), jnp.float32),
            jax.ShapeDtypeStruct((B, _NUM_ANCHORS * _S, _NUM_CLASSES),
                                 jnp.float32),
        ],
    )(x)
    return (boxes, conf, cls)


# speedup vs baseline: 9.2407x; 9.2407x over previous
import jax
import jax.numpy as jnp
import numpy as np
from jax.experimental import pallas as pl

_NUM_ANCHORS = 3
_NUM_CLASSES = 80
_STRIDE = 16.0
_G = 64
_S = _G * _G
_CH = _NUM_CLASSES + 5
_C = _NUM_ANCHORS * _CH
_N = _NUM_ANCHORS * _S
_ANCHORS = ((0.12, 0.16), (0.30, 0.38), (0.72, 0.55))
_AW16 = tuple(float(np.float32(a[0]) * _G * _STRIDE) for a in _ANCHORS)
_AH16 = tuple(float(np.float32(a[1]) * _G * _STRIDE) for a in _ANCHORS)


def _yolo_body(x_ref, boxes_ref, conf_ref, cls_ref):
    xs = x_ref[0].reshape(_S, _C)

    lane = jax.lax.broadcasted_iota(jnp.int32, (1, _C), 1)
    cc = lane - _CH * (lane // _CH)
    is_wh = (cc == 2) | (cc == 3)
    e = jnp.exp(-xs)
    act = 1.0 / jnp.where(is_wh, e, 1.0 + e)

    act_t = act.T

    cls_ref[0] = jnp.concatenate(
        [act_t[a * _CH + 5:a * _CH + _CH] for a in range(_NUM_ANCHORS)],
        axis=1)

    conf_ref[0] = jnp.concatenate(
        [act_t[a * _CH + 4:a * _CH + 5] for a in range(_NUM_ANCHORS)],
        axis=1)

    raw = jnp.concatenate(
        [act_t[a * _CH:a * _CH + 4] for a in range(_NUM_ANCHORS)],
        axis=1)
    k = jax.lax.broadcasted_iota(jnp.int32, (4, _N), 0)
    li = jax.lax.broadcasted_iota(jnp.int32, (4, _N), 1)
    s = li & (_S - 1)
    gx = (s & (_G - 1)).astype(jnp.float32)
    gy = (s >> 6).astype(jnp.float32)
    offs = jnp.where(k == 0, gx, jnp.where(k == 1, gy, 0.0))
    a_of = li >> 12
    aw = jnp.where(a_of == 0, _AW16[0],
                   jnp.where(a_of == 1, _AW16[1], _AW16[2]))
    ah = jnp.where(a_of == 0, _AH16[0],
                   jnp.where(a_of == 1, _AH16[1], _AH16[2]))
    scale = jnp.where(k < 2, _STRIDE, jnp.where(k == 2, aw, ah))
    boxes_ref[0] = (raw + offs) * scale


def kernel(x):
    B = x.shape[0]
    xt = jnp.transpose(x, (0, 2, 3, 1))

    boxes_t, conf_t, cls_t = pl.pallas_call(
        _yolo_body,
        grid=(B,),
        in_specs=[pl.BlockSpec((1, _G, _G, _C), lambda b: (b, 0, 0, 0))],
        out_specs=[
            pl.BlockSpec((1, 4, _N), lambda b: (b, 0, 0)),
            pl.BlockSpec((1, 1, _N), lambda b: (b, 0, 0)),
            pl.BlockSpec((1, _NUM_CLASSES, _N), lambda b: (b, 0, 0)),
        ],
        out_shape=[
            jax.ShapeDtypeStruct((B, 4, _N), jnp.float32),
            jax.ShapeDtypeStruct((B, 1, _N), jnp.float32),
            jax.ShapeDtypeStruct((B, _NUM_CLASSES, _N), jnp.float32),
        ],
    )(xt)

    out_boxes = jnp.transpose(boxes_t, (0, 2, 1))
    out_conf = jnp.transpose(conf_t, (0, 2, 1))
    out_cls = jnp.transpose(cls_t, (0, 2, 1))
    return (out_boxes, out_conf, out_cls)


# speedup vs baseline: 9.5658x; 1.0352x over previous
import jax
import jax.numpy as jnp
import numpy as np
from jax.experimental import pallas as pl

_NUM_ANCHORS = 3
_NUM_CLASSES = 80
_STRIDE = 16.0
_G = 64
_S = _G * _G
_CH = _NUM_CLASSES + 5
_C = _NUM_ANCHORS * _CH
_N = _NUM_ANCHORS * _S
_ANCHORS = ((0.12, 0.16), (0.30, 0.38), (0.72, 0.55))
_AW16 = tuple(float(np.float32(a[0]) * _G * _STRIDE) for a in _ANCHORS)
_AH16 = tuple(float(np.float32(a[1]) * _G * _STRIDE) for a in _ANCHORS)


def _yolo_body(x_ref, boxes_ref, conf_ref, cls_ref):
    for bb in range(x_ref.shape[0]):
        _yolo_one(x_ref, boxes_ref, conf_ref, cls_ref, bb)


def _yolo_one(x_ref, boxes_ref, conf_ref, cls_ref, bb):
    xs = x_ref[bb].reshape(_S, _C)

    lane = jax.lax.broadcasted_iota(jnp.int32, (1, _C), 1)
    cc = lane - _CH * (lane // _CH)
    is_wh = (cc == 2) | (cc == 3)
    e = jnp.exp(-xs)
    act = 1.0 / jnp.where(is_wh, e, 1.0 + e)

    act_t = act.T

    cls_ref[bb] = jnp.concatenate(
        [act_t[a * _CH + 5:a * _CH + _CH] for a in range(_NUM_ANCHORS)],
        axis=1)

    conf_ref[bb] = jnp.concatenate(
        [act_t[a * _CH + 4:a * _CH + 5] for a in range(_NUM_ANCHORS)],
        axis=1)

    raw = jnp.concatenate(
        [act_t[a * _CH:a * _CH + 4] for a in range(_NUM_ANCHORS)],
        axis=1)
    k = jax.lax.broadcasted_iota(jnp.int32, (4, _N), 0)
    li = jax.lax.broadcasted_iota(jnp.int32, (4, _N), 1)
    s = li & (_S - 1)
    gx = (s & (_G - 1)).astype(jnp.float32)
    gy = (s >> 6).astype(jnp.float32)
    offs = jnp.where(k == 0, gx, jnp.where(k == 1, gy, 0.0))
    a_of = li >> 12
    aw = jnp.where(a_of == 0, _AW16[0],
                   jnp.where(a_of == 1, _AW16[1], _AW16[2]))
    ah = jnp.where(a_of == 0, _AH16[0],
                   jnp.where(a_of == 1, _AH16[1], _AH16[2]))
    scale = jnp.where(k < 2, _STRIDE, jnp.where(k == 2, aw, ah))
    boxes_ref[bb] = (raw + offs) * scale


_BB = 2


def kernel(x):
    B = x.shape[0]
    xt = jnp.transpose(x, (0, 2, 3, 1))

    boxes_t, conf_t, cls_t = pl.pallas_call(
        _yolo_body,
        grid=(B // _BB,),
        in_specs=[pl.BlockSpec((_BB, _G, _G, _C), lambda b: (b, 0, 0, 0))],
        out_specs=[
            pl.BlockSpec((_BB, 4, _N), lambda b: (b, 0, 0)),
            pl.BlockSpec((_BB, 1, _N), lambda b: (b, 0, 0)),
            pl.BlockSpec((_BB, _NUM_CLASSES, _N), lambda b: (b, 0, 0)),
        ],
        out_shape=[
            jax.ShapeDtypeStruct((B, 4, _N), jnp.float32),
            jax.ShapeDtypeStruct((B, 1, _N), jnp.float32),
            jax.ShapeDtypeStruct((B, _NUM_CLASSES, _N), jnp.float32),
        ],
    )(xt)

    out_boxes = jnp.transpose(boxes_t, (0, 2, 1))
    out_conf = jnp.transpose(conf_t, (0, 2, 1))
    out_cls = jnp.transpose(cls_t, (0, 2, 1))
    return (out_boxes, out_conf, out_cls)
